# baseline (speedup 1.0000x reference)
import functools

import jax
import jax.numpy as jnp
from jax import lax
from jax.experimental import pallas as pl
from jax.experimental.pallas import tpu as pltpu
from jax.experimental.pallas import tpu_sc as plsc

N_HID = 60
N_SYM = 26
N_PAIR = N_SYM * N_SYM
TBL_PAD = 688
BATCH = 16384


def _table_body(a_ref, b_ref, emb_a_ref, emb_b_ref, wc_ref, bc_ref, wo_ref,
                bo_ref, out_ref, idx_ref):
    idx_ref[...] = a_ref[...] * N_SYM + b_ref[...]
    ea = jnp.maximum(emb_a_ref[...], 0.0)
    eb = jnp.maximum(emb_b_ref[...], 0.0)

    p_idx = lax.broadcasted_iota(jnp.int32, (TBL_PAD, N_SYM), 0) // N_SYM
    q_idx = lax.broadcasted_iota(jnp.int32, (TBL_PAD, N_SYM), 0) % N_SYM
    sym = lax.broadcasted_iota(jnp.int32, (TBL_PAD, N_SYM), 1)
    ph = jnp.where(p_idx == sym, 1.0, 0.0)
    qh = jnp.where(q_idx == sym, 1.0, 0.0)

    cat = jnp.concatenate(
        (jnp.dot(ph, ea, preferred_element_type=jnp.float32),
         jnp.dot(qh, eb, preferred_element_type=jnp.float32)),
        axis=1)
    h = lax.dot_general(cat, wc_ref[...], (((1,), (1,)), ((), ())),
                        preferred_element_type=jnp.float32)
    h = jnp.maximum(h + bc_ref[...], 0.0)
    t = lax.dot_general(wo_ref[...], h, (((1,), (1,)), ((), ())),
                        preferred_element_type=jnp.float32)
    out_ref[...] = (t + bo_ref[...]).reshape(TBL_PAD)


def _build_table(a, b, emb_a, emb_b, w_comp, b_comp, w_out, b_out):
    return pl.pallas_call(
        _table_body,
        out_shape=(jax.ShapeDtypeStruct((TBL_PAD,), jnp.float32),
                   jax.ShapeDtypeStruct((BATCH,), jnp.int32)),
    )(a, b, emb_a, emb_b, w_comp, b_comp, w_out, b_out)


_NCORES = 2
_NSUB = 16
_NW = _NCORES * _NSUB
_PER_W = BATCH // _NW
_LANES = 16


def _gather_body(tbl_hbm, idx_hbm, out_hbm, tbl_v, idx_v, out_v,
                 sem_t, sem_i):
    wid = lax.axis_index("s")
    base = wid * _PER_W
    ct = pltpu.async_copy(tbl_hbm, tbl_v, sem_t)
    ci = pltpu.async_copy(idx_hbm.at[pl.ds(base, _PER_W)], idx_v, sem_i)
    ct.wait()
    ci.wait()
    for i in range(_PER_W // _LANES):
        sl = pl.ds(i * _LANES, _LANES)
        out_v[sl] = plsc.load_gather(tbl_v, [idx_v[sl]])
    pltpu.sync_copy(out_v, out_hbm.at[pl.ds(base, _PER_W)])


@functools.lru_cache(maxsize=1)
def _make_gather():
    return pl.kernel(
        _gather_body,
        out_type=jax.ShapeDtypeStruct((BATCH,), jnp.float32),
        mesh=plsc.VectorSubcoreMesh(core_axis_name="c", subcore_axis_name="s",
                                    num_cores=_NCORES, num_subcores=_NSUB),
        compiler_params=pltpu.CompilerParams(needs_layout_passes=False),
        scratch_types=[
            pltpu.VMEM((TBL_PAD,), jnp.float32),
            pltpu.VMEM((_PER_W,), jnp.int32),
            pltpu.VMEM((_PER_W,), jnp.float32),
            pltpu.SemaphoreType.DMA,
            pltpu.SemaphoreType.DMA,
        ],
    )


@jax.jit
def kernel(a, b, emb_a, emb_b, W_comp, b_comp, W_out, b_out):
    table, idx = _build_table(a, b, emb_a, emb_b, W_comp,
                              b_comp.reshape(1, N_HID), W_out,
                              b_out.reshape(1, 1))
    out = _make_gather()(table, idx)
    return out.reshape(BATCH, 1)


# speedup vs baseline: 1.0971x; 1.0971x over previous
import functools

import jax
import jax.numpy as jnp
from jax import lax
from jax.experimental import pallas as pl
from jax.experimental.pallas import tpu as pltpu
from jax.experimental.pallas import tpu_sc as plsc

N_HID = 60
N_SYM = 26
N_PAIR = N_SYM * N_SYM
TBL_PAD = 688
BATCH = 16384


def _table_body(a_ref, b_ref, emb_a_ref, emb_b_ref, wc_ref, bc_ref, wo_ref,
                bo_ref, out_ref, idx_ref):
    idx_ref[...] = a_ref[...] * N_SYM + b_ref[...]
    ea = jnp.maximum(emb_a_ref[...], 0.0)
    eb = jnp.maximum(emb_b_ref[...], 0.0)

    p_idx = lax.broadcasted_iota(jnp.int32, (TBL_PAD, N_SYM), 0) // N_SYM
    q_idx = lax.broadcasted_iota(jnp.int32, (TBL_PAD, N_SYM), 0) % N_SYM
    sym = lax.broadcasted_iota(jnp.int32, (TBL_PAD, N_SYM), 1)
    ph = jnp.where(p_idx == sym, 1.0, 0.0)
    qh = jnp.where(q_idx == sym, 1.0, 0.0)

    cat = jnp.concatenate(
        (jnp.dot(ph, ea, preferred_element_type=jnp.float32),
         jnp.dot(qh, eb, preferred_element_type=jnp.float32)),
        axis=1)
    h = lax.dot_general(cat, wc_ref[...], (((1,), (1,)), ((), ())),
                        preferred_element_type=jnp.float32)
    h = jnp.maximum(h + bc_ref[...], 0.0)
    t = lax.dot_general(wo_ref[...], h, (((1,), (1,)), ((), ())),
                        preferred_element_type=jnp.float32)
    out_ref[...] = (t + bo_ref[...]).reshape(TBL_PAD)


def _build_table(a, b, emb_a, emb_b, w_comp, b_comp, w_out, b_out):
    return pl.pallas_call(
        _table_body,
        out_shape=(jax.ShapeDtypeStruct((TBL_PAD,), jnp.float32),
                   jax.ShapeDtypeStruct((BATCH,), jnp.int32)),
    )(a, b, emb_a, emb_b, w_comp, b_comp, w_out, b_out)


_NCORES = 1
_NSUB = 16
_NW = _NCORES * _NSUB
_PER_W = BATCH // _NW
_LANES = 16


def _gather_body(tbl_hbm, idx_hbm, out_hbm, tbl_v, idx_v, out_v,
                 sem_t, sem_i):
    wid = lax.axis_index("s")
    base = wid * _PER_W
    ct = pltpu.async_copy(tbl_hbm, tbl_v, sem_t)
    ci = pltpu.async_copy(idx_hbm.at[pl.ds(base, _PER_W)], idx_v, sem_i)
    ct.wait()
    ci.wait()
    for i in range(_PER_W // _LANES):
        sl = pl.ds(i * _LANES, _LANES)
        out_v[sl] = plsc.load_gather(tbl_v, [idx_v[sl]])
    pltpu.sync_copy(out_v, out_hbm.at[pl.ds(base, _PER_W)])


@functools.lru_cache(maxsize=1)
def _make_gather():
    return pl.kernel(
        _gather_body,
        out_type=jax.ShapeDtypeStruct((BATCH,), jnp.float32),
        mesh=plsc.VectorSubcoreMesh(core_axis_name="c", subcore_axis_name="s",
                                    num_cores=_NCORES, num_subcores=_NSUB),
        compiler_params=pltpu.CompilerParams(needs_layout_passes=False),
        scratch_types=[
            pltpu.VMEM((TBL_PAD,), jnp.float32),
            pltpu.VMEM((_PER_W,), jnp.int32),
            pltpu.VMEM((_PER_W,), jnp.float32),
            pltpu.SemaphoreType.DMA,
            pltpu.SemaphoreType.DMA,
        ],
    )


@jax.jit
def kernel(a, b, emb_a, emb_b, W_comp, b_comp, W_out, b_out):
    table, idx = _build_table(a, b, emb_a, emb_b, W_comp,
                              b_comp.reshape(1, N_HID), W_out,
                              b_out.reshape(1, 1))
    out = _make_gather()(table, idx)
    return out.reshape(BATCH, 1)
